# baseline (speedup 1.0000x reference)
import functools

import jax
import jax.numpy as jnp
from jax import lax
from jax.experimental import pallas as pl
from jax.experimental.pallas import tpu as pltpu
from jax.experimental.pallas import tpu_sc as plsc

VOCAB = 1000000
SEQ = 200
EMBED = 64
BATCH = 4096

NC = 2
NS = 16
NW = NC * NS
ROWS_PER_W = BATCH * SEQ // NW
LANES = 16
VPR = EMBED // LANES

BLK = 256
PBLK = BLK // 2
NBLK = ROWS_PER_W // BLK
POS_REP = 448

_mesh = plsc.VectorSubcoreMesh(core_axis_name="c", subcore_axis_name="s")


@functools.partial(
    pl.kernel,
    out_type=jax.ShapeDtypeStruct((BATCH * SEQ // 2, 2 * EMBED),
                                  jnp.float32),
    mesh=_mesh,
    scratch_types=[
        pltpu.VMEM((POS_REP, EMBED), jnp.float32),
        [pltpu.VMEM((BLK, 2 * EMBED), jnp.float32) for _ in range(2)],
        [pltpu.VMEM((BLK,), jnp.int32) for _ in range(2)],
        [pltpu.SemaphoreType.DMA for _ in range(2)],
        [pltpu.SemaphoreType.DMA for _ in range(2)],
        [pltpu.SemaphoreType.DMA for _ in range(2)],
    ],
)
def _gather_add(seq_hbm, tokw_hbm, pos_hbm, out_hbm, pos_v, bufs, idxs,
                isems, gsems, ssems):
    wid = lax.axis_index("s") * NC + lax.axis_index("c")
    base = wid * ROWS_PER_W
    pbase = base // 2

    for r0 in range(0, POS_REP, SEQ):
        n = min(SEQ, POS_REP - r0)
        pltpu.sync_copy(pos_hbm.at[pl.ds(0, n)], pos_v.at[pl.ds(r0, n)])

    SUBS = ((0, 128), (128, 128))

    def start_idx(b, slot):
        pltpu.async_copy(
            seq_hbm.at[pl.ds(pl.multiple_of(base + b * BLK, BLK), BLK)],
            idxs[slot], isems[slot])

    def wait_idx(slot):
        pltpu.make_async_copy(
            seq_hbm.at[pl.ds(0, BLK)], idxs[slot], isems[slot]).wait()

    def start_gather(slot):
        for o, n in SUBS:
            pltpu.async_copy(
                tokw_hbm.at[idxs[slot].at[pl.ds(o, n)]],
                bufs[slot].at[pl.ds(o, n)], gsems[slot])

    def wait_gather(slot):
        for o, n in SUBS:
            pltpu.make_async_copy(
                tokw_hbm.at[idxs[slot].at[pl.ds(o, n)]],
                bufs[slot].at[pl.ds(o, n)], gsems[slot]).wait()

    def start_scatter(b, slot):
        pltpu.async_copy(
            bufs[slot].at[pl.ds(0, PBLK)],
            out_hbm.at[pl.ds(pl.multiple_of(pbase + b * PBLK, PBLK), PBLK)],
            ssems[slot])

    def wait_scatter(slot):
        pltpu.make_async_copy(
            bufs[slot].at[pl.ds(0, PBLK)],
            out_hbm.at[pl.ds(0, PBLK)], ssems[slot]).wait()

    start_idx(0, 0)
    wait_idx(0)
    start_gather(0)
    start_idx(1, 1)

    def step(b, s):
        o = 1 - s
        wait_gather(s)

        @pl.when(b + 2 < NBLK)
        def _():
            start_idx(b + 2, s)

        @pl.when(b + 1 < NBLK)
        def _():
            wait_idx(o)

            @pl.when(b >= 1)
            def _():
                wait_scatter(o)
            start_gather(o)

        p0 = lax.rem(b * BLK, SEQ)
        buf = bufs[s]

        def pair_rows(i, carry):
            for k in range(VPR):
                plsc.addupdate(
                    buf.at[i, pl.ds(k * LANES, LANES)],
                    pos_v[p0 + i, pl.ds(k * LANES, LANES)])
            for k in range(VPR):
                buf[i, pl.ds(EMBED + k * LANES, LANES)] = (
                    buf[i + PBLK, pl.ds(k * LANES, LANES)]
                    + pos_v[p0 + PBLK + i, pl.ds(k * LANES, LANES)])
            return carry

        lax.fori_loop(0, PBLK, pair_rows, 0, unroll=2)

        start_scatter(b, s)

    def group(g, carry):
        step(2 * g, 0)
        step(2 * g + 1, 1)
        return carry

    lax.fori_loop(0, NBLK // 2, group, 0)

    wait_scatter(0)
    wait_scatter(1)


def kernel(sequence, token_table, pos_table):
    seq_flat = sequence.reshape(-1).astype(jnp.int32)
    tokw = jnp.concatenate([token_table, token_table], axis=1)
    packed = _gather_add(seq_flat, tokw, pos_table)
    out = packed.reshape(BATCH * SEQ // BLK, PBLK, 2, EMBED)
    out = out.transpose(0, 2, 1, 3)
    return out.reshape(BATCH, SEQ, EMBED)


# speedup vs baseline: 1.3420x; 1.3420x over previous
import functools

import jax
import jax.numpy as jnp
from jax import lax
from jax.experimental import pallas as pl
from jax.experimental.pallas import tpu as pltpu
from jax.experimental.pallas import tpu_sc as plsc

VOCAB = 1000000
SEQ = 200
EMBED = 64
BATCH = 4096

NC = 2
NS = 16
NW = NC * NS
BPW = BATCH // NW
LANES = 16
VPR = EMBED // LANES

SCH = 2
NCHUNK = SEQ // SCH
ROWS = SCH * BPW
OROWS = SCH * EMBED

_mesh = plsc.VectorSubcoreMesh(core_axis_name="c", subcore_axis_name="s")


@functools.partial(
    pl.kernel,
    out_type=jax.ShapeDtypeStruct((SEQ * EMBED, BATCH), jnp.float32),
    mesh=_mesh,
    compiler_params=pltpu.CompilerParams(needs_layout_passes=False),
    scratch_types=[
        pltpu.VMEM((SEQ, EMBED), jnp.float32),
        [pltpu.VMEM((ROWS, 2 * EMBED), jnp.float32) for _ in range(2)],
        pltpu.VMEM((OROWS, BATCH // NW), jnp.float32),
        [pltpu.VMEM((SCH, BPW), jnp.int32) for _ in range(2)],
        [pltpu.SemaphoreType.DMA for _ in range(2)],
        [pltpu.SemaphoreType.DMA for _ in range(2)],
        pltpu.SemaphoreType.DMA,
    ],
)
def _gather_add(seqT_hbm, tokw_hbm, pos_hbm, v_hbm, pos_v, bufs, outb,
                idxs, isems, gsems, ssem):
    wid = lax.axis_index("s") * NC + lax.axis_index("c")
    col0 = pl.multiple_of(wid * BPW, BPW)

    pltpu.sync_copy(pos_hbm, pos_v)

    def start_idx(c, slot):
        pltpu.async_copy(
            seqT_hbm.at[pl.ds(pl.multiple_of(c * SCH, SCH), SCH),
                        pl.ds(col0, BPW)],
            idxs[slot], isems[slot])

    def wait_idx(slot):
        pltpu.make_async_copy(
            seqT_hbm.at[pl.ds(0, SCH), pl.ds(col0, BPW)],
            idxs[slot], isems[slot]).wait()

    def start_gather(slot):
        for j in range(SCH):
            pltpu.async_copy(
                tokw_hbm.at[idxs[slot].at[j]],
                bufs[slot].at[pl.ds(j * BPW, BPW)], gsems[slot])

    def wait_gather(slot):
        for j in range(SCH):
            pltpu.make_async_copy(
                tokw_hbm.at[idxs[slot].at[j]],
                bufs[slot].at[pl.ds(j * BPW, BPW)], gsems[slot]).wait()

    def start_scatter(c):
        pltpu.async_copy(
            outb,
            v_hbm.at[pl.ds(pl.multiple_of(c * OROWS, OROWS), OROWS),
                     pl.ds(col0, BPW)],
            ssem)

    def wait_scatter():
        pltpu.make_async_copy(
            outb, v_hbm.at[pl.ds(0, OROWS), pl.ds(col0, BPW)], ssem).wait()

    iota = lax.iota(jnp.int32, LANES)

    start_idx(0, 0)
    wait_idx(0)
    start_gather(0)
    start_idx(1, 1)

    def step(c, s):
        o = 1 - s
        wait_gather(s)

        @pl.when(c + 2 < NCHUNK)
        def _():
            start_idx(c + 2, s)

        @pl.when(c + 1 < NCHUNK)
        def _():
            wait_idx(o)
            start_gather(o)

        @pl.when(c >= 1)
        def _():
            wait_scatter()

        buf = bufs[s]

        def row(r, carry):
            s_l = r // BPW
            b_l = lax.rem(r, BPW)
            gs = c * SCH + s_l
            for k in range(VPR):
                v = (buf[r, pl.ds(k * LANES, LANES)]
                     + pos_v[gs, pl.ds(k * LANES, LANES)])
                rows_k = s_l * EMBED + k * LANES + iota
                plsc.store_scatter(
                    outb, [rows_k, jnp.full((LANES,), b_l, jnp.int32)], v)
            return carry

        lax.fori_loop(0, ROWS, row, 0, unroll=2)

        start_scatter(c)

    def group(g, carry):
        step(2 * g, 0)
        step(2 * g + 1, 1)
        return carry

    lax.fori_loop(0, NCHUNK // 2, group, 0)

    wait_scatter()


def kernel(sequence, token_table, pos_table):
    seqT = sequence.T.astype(jnp.int32)
    tokw = jnp.pad(token_table, ((0, 0), (0, 2 * EMBED - EMBED)))
    v = _gather_add(seqT, tokw, pos_table)
    return v.reshape(SEQ, EMBED, BATCH).transpose(2, 0, 1)


# speedup vs baseline: 1.3434x; 1.0011x over previous
import functools

import jax
import jax.numpy as jnp
from jax import lax
from jax.experimental import pallas as pl
from jax.experimental.pallas import tpu as pltpu
from jax.experimental.pallas import tpu_sc as plsc

VOCAB = 1000000
SEQ = 200
EMBED = 64
BATCH = 4096

NC = 2
NS = 16
NW = NC * NS
BPW = BATCH // NW
LANES = 16
VPR = EMBED // LANES

SCH = 2
NCHUNK = SEQ // SCH
ROWS = SCH * BPW
OROWS = SCH * EMBED

_mesh = plsc.VectorSubcoreMesh(core_axis_name="c", subcore_axis_name="s")


@functools.partial(
    pl.kernel,
    out_type=jax.ShapeDtypeStruct((SEQ * EMBED, BATCH), jnp.float32),
    mesh=_mesh,
    compiler_params=pltpu.CompilerParams(needs_layout_passes=False),
    scratch_types=[
        pltpu.VMEM((SEQ, EMBED), jnp.float32),
        [pltpu.VMEM((ROWS, 2 * EMBED), jnp.float32) for _ in range(2)],
        pltpu.VMEM((OROWS, BPW + 1), jnp.float32),
        [pltpu.VMEM((SCH, BPW), jnp.int32) for _ in range(2)],
        [pltpu.SemaphoreType.DMA for _ in range(2)],
        [pltpu.SemaphoreType.DMA for _ in range(2)],
        pltpu.SemaphoreType.DMA,
    ],
)
def _gather_add(seqT_hbm, tokw_hbm, pos_hbm, v_hbm, pos_v, bufs, outb,
                idxs, isems, gsems, ssem):
    wid = lax.axis_index("s") * NC + lax.axis_index("c")
    col0 = pl.multiple_of(wid * BPW, BPW)

    pltpu.sync_copy(pos_hbm, pos_v)

    def start_idx(c, slot):
        pltpu.async_copy(
            seqT_hbm.at[pl.ds(pl.multiple_of(c * SCH, SCH), SCH),
                        pl.ds(col0, BPW)],
            idxs[slot], isems[slot])

    def wait_idx(slot):
        pltpu.make_async_copy(
            seqT_hbm.at[pl.ds(0, SCH), pl.ds(col0, BPW)],
            idxs[slot], isems[slot]).wait()

    def start_gather(slot):
        for j in range(SCH):
            pltpu.async_copy(
                tokw_hbm.at[idxs[slot].at[j]],
                bufs[slot].at[pl.ds(j * BPW, BPW)], gsems[slot])

    def wait_gather(slot):
        for j in range(SCH):
            pltpu.make_async_copy(
                tokw_hbm.at[idxs[slot].at[j]],
                bufs[slot].at[pl.ds(j * BPW, BPW)], gsems[slot]).wait()

    def start_scatter(c):
        pltpu.async_copy(
            outb.at[:, pl.ds(0, BPW)],
            v_hbm.at[pl.ds(pl.multiple_of(c * OROWS, OROWS), OROWS),
                     pl.ds(col0, BPW)],
            ssem)

    def wait_scatter():
        pltpu.make_async_copy(
            outb.at[:, pl.ds(0, BPW)],
            v_hbm.at[pl.ds(0, OROWS), pl.ds(col0, BPW)], ssem).wait()

    iota = lax.iota(jnp.int32, LANES)

    start_idx(0, 0)
    wait_idx(0)
    start_gather(0)
    start_idx(1, 1)

    def step(c, s):
        o = 1 - s
        wait_gather(s)

        @pl.when(c + 2 < NCHUNK)
        def _():
            start_idx(c + 2, s)

        @pl.when(c + 1 < NCHUNK)
        def _():
            wait_idx(o)
            start_gather(o)

        @pl.when(c >= 1)
        def _():
            wait_scatter()

        buf = bufs[s]

        def row(r, carry):
            s_l = r // BPW
            b_l = lax.rem(r, BPW)
            gs = c * SCH + s_l
            for k in range(VPR):
                v = (buf[r, pl.ds(k * LANES, LANES)]
                     + pos_v[gs, pl.ds(k * LANES, LANES)])
                rows_k = s_l * EMBED + k * LANES + iota
                plsc.store_scatter(
                    outb, [rows_k, jnp.full((LANES,), b_l, jnp.int32)], v)
            return carry

        lax.fori_loop(0, ROWS, row, 0, unroll=2)

        start_scatter(c)

    def group(g, carry):
        step(2 * g, 0)
        step(2 * g + 1, 1)
        return carry

    lax.fori_loop(0, NCHUNK // 2, group, 0)

    wait_scatter()


def kernel(sequence, token_table, pos_table):
    seqT = sequence.T.astype(jnp.int32)
    tokw = jnp.pad(token_table, ((0, 0), (0, 2 * EMBED - EMBED)))
    v = _gather_add(seqT, tokw, pos_table)
    return v.reshape(SEQ, EMBED, BATCH).transpose(2, 0, 1)


# speedup vs baseline: 1.3841x; 1.0302x over previous
import functools

import jax
import jax.numpy as jnp
from jax import lax
from jax.experimental import pallas as pl
from jax.experimental.pallas import tpu as pltpu
from jax.experimental.pallas import tpu_sc as plsc

VOCAB = 1000000
SEQ = 200
EMBED = 64
BATCH = 4096

NC = 2
NS = 16
NW = NC * NS
BPW = BATCH // NW
LANES = 16
VPR = EMBED // LANES

SCH = 2
NCHUNK = SEQ // SCH
ROWS = SCH * BPW
OROWS = SCH * EMBED

_mesh = plsc.VectorSubcoreMesh(core_axis_name="c", subcore_axis_name="s")


@functools.partial(
    pl.kernel,
    out_type=jax.ShapeDtypeStruct((SEQ * EMBED, BATCH), jnp.float32),
    mesh=_mesh,
    compiler_params=pltpu.CompilerParams(needs_layout_passes=False),
    scratch_types=[
        pltpu.VMEM((SEQ, EMBED), jnp.float32),
        [pltpu.VMEM((ROWS, 2 * EMBED), jnp.float32) for _ in range(2)],
        pltpu.VMEM((OROWS, BPW + 1), jnp.float32),
        [pltpu.VMEM((SCH, BPW), jnp.int32) for _ in range(2)],
        [pltpu.SemaphoreType.DMA for _ in range(2)],
        [pltpu.SemaphoreType.DMA for _ in range(2)],
        pltpu.SemaphoreType.DMA,
    ],
)
def _gather_add(seqT_hbm, tokw_hbm, pos_hbm, v_hbm, pos_v, bufs, outb,
                idxs, isems, gsems, ssem):
    wid = lax.axis_index("s") * NC + lax.axis_index("c")
    col0 = pl.multiple_of(wid * BPW, BPW)

    pltpu.sync_copy(pos_hbm, pos_v)

    def start_idx(c, slot):
        pltpu.async_copy(
            seqT_hbm.at[pl.ds(pl.multiple_of(c * SCH, SCH), SCH),
                        pl.ds(col0, BPW)],
            idxs[slot], isems[slot])

    def wait_idx(slot):
        pltpu.make_async_copy(
            seqT_hbm.at[pl.ds(0, SCH), pl.ds(col0, BPW)],
            idxs[slot], isems[slot]).wait()

    def start_gather(slot):
        for j in range(SCH):
            pltpu.async_copy(
                tokw_hbm.at[idxs[slot].at[j]],
                bufs[slot].at[pl.ds(j * BPW, BPW)], gsems[slot])

    def wait_gather(slot):
        for j in range(SCH):
            pltpu.make_async_copy(
                tokw_hbm.at[idxs[slot].at[j]],
                bufs[slot].at[pl.ds(j * BPW, BPW)], gsems[slot]).wait()

    def start_scatter(c):
        pltpu.async_copy(
            outb.at[:, pl.ds(0, BPW)],
            v_hbm.at[pl.ds(pl.multiple_of(c * OROWS, OROWS), OROWS),
                     pl.ds(col0, BPW)],
            ssem)

    def wait_scatter():
        pltpu.make_async_copy(
            outb.at[:, pl.ds(0, BPW)],
            v_hbm.at[pl.ds(0, OROWS), pl.ds(col0, BPW)], ssem).wait()

    iota = lax.iota(jnp.int32, LANES)

    start_idx(0, 0)
    wait_idx(0)
    start_gather(0)
    start_idx(1, 1)

    def step(c, s):
        o = 1 - s
        wait_gather(s)

        @pl.when(c + 2 < NCHUNK)
        def _():
            start_idx(c + 2, s)

        @pl.when(c + 1 < NCHUNK)
        def _():
            wait_idx(o)
            start_gather(o)

        @pl.when(c >= 1)
        def _():
            wait_scatter()

        buf = bufs[s]

        for s_l in range(SCH):
            gs = c * SCH + s_l
            pos_k = [pos_v[gs, pl.ds(k * LANES, LANES)] for k in range(VPR)]
            row_k = [s_l * EMBED + k * LANES + iota for k in range(VPR)]

            def brow(b_l, colv, s_l=s_l, pos_k=pos_k, row_k=row_k):
                r = s_l * BPW + b_l
                for k in range(VPR):
                    v = buf[r, pl.ds(k * LANES, LANES)] + pos_k[k]
                    plsc.store_scatter(outb, [row_k[k], colv], v)
                return colv + 1

            lax.fori_loop(0, BPW, brow,
                          jnp.zeros((LANES,), jnp.int32), unroll=4)

        start_scatter(c)

    def group(g, carry):
        step(2 * g, 0)
        step(2 * g + 1, 1)
        return carry

    lax.fori_loop(0, NCHUNK // 2, group, 0)

    wait_scatter()


def kernel(sequence, token_table, pos_table):
    seqT = sequence.T.astype(jnp.int32)
    tokw = jnp.pad(token_table, ((0, 0), (0, 2 * EMBED - EMBED)))
    v = _gather_add(seqT, tokw, pos_table)
    return v.reshape(SEQ, EMBED, BATCH).transpose(2, 0, 1)


# speedup vs baseline: 1.6491x; 1.1915x over previous
import functools

import jax
import jax.numpy as jnp
from jax import lax
from jax.experimental import pallas as pl
from jax.experimental.pallas import tpu as pltpu
from jax.experimental.pallas import tpu_sc as plsc

VOCAB = 1000000
SEQ = 200
EMBED = 64
BATCH = 4096

NC = 2
NS = 16
NW = NC * NS
SEQS_PER_W = BATCH // NW
ROWS_PER_W = SEQS_PER_W * SEQ
LANES = 16
VPR = EMBED // LANES

_mesh = plsc.VectorSubcoreMesh(core_axis_name="c", subcore_axis_name="s")


@functools.partial(
    pl.kernel,
    out_type=jax.ShapeDtypeStruct((BATCH * SEQ, EMBED), jnp.float32),
    mesh=_mesh,
    compiler_params=pltpu.CompilerParams(use_tc_tiling_on_sc=False),
    scratch_types=[
        pltpu.VMEM((ROWS_PER_W,), jnp.int32),
        pltpu.VMEM((SEQ, EMBED), jnp.float32),
        pltpu.VMEM((SEQ, EMBED), jnp.float32),
        pltpu.SemaphoreType.DMA,
    ],
)
def _seq_embed(seq_hbm, tok_hbm, pos_hbm, out_hbm, idx_v, pos_v, buf, sem):
    wid = lax.axis_index("s") * NC + lax.axis_index("c")
    base = wid * ROWS_PER_W

    pltpu.sync_copy(seq_hbm.at[pl.ds(base, ROWS_PER_W)], idx_v)
    pltpu.sync_copy(pos_hbm, pos_v)

    def chunk(c, carry):
        row0 = c * SEQ
        g0 = pltpu.async_copy(
            tok_hbm.at[idx_v.at[pl.ds(row0, 128)]], buf.at[pl.ds(0, 128)], sem)
        g1 = pltpu.async_copy(
            tok_hbm.at[idx_v.at[pl.ds(row0 + 128, SEQ - 128)]],
            buf.at[pl.ds(128, SEQ - 128)], sem)
        g0.wait()
        g1.wait()

        def add_row(j, carry2):
            for k in range(VPR):
                plsc.addupdate(
                    buf.at[j, pl.ds(k * LANES, LANES)],
                    pos_v[j, pl.ds(k * LANES, LANES)])
            return carry2

        lax.fori_loop(0, SEQ, add_row, 0, unroll=2)

        pltpu.sync_copy(buf, out_hbm.at[pl.ds(base + row0, SEQ)])
        return carry

    lax.fori_loop(0, SEQS_PER_W, chunk, 0)


def kernel(sequence, token_table, pos_table):
    seq_flat = sequence.reshape(-1).astype(jnp.int32)
    out = _seq_embed(seq_flat, token_table, pos_table)
    return out.reshape(BATCH, SEQ, EMBED)


# speedup vs baseline: 1.7029x; 1.0326x over previous
import functools

import jax
import jax.numpy as jnp
from jax import lax
from jax.experimental import pallas as pl
from jax.experimental.pallas import tpu as pltpu
from jax.experimental.pallas import tpu_sc as plsc

VOCAB = 1000000
SEQ = 200
EMBED = 64
BATCH = 4096

NC = 2
NS = 16
NW = NC * NS
SEQS_PER_W = BATCH // NW
ROWS_PER_W = SEQS_PER_W * SEQ
LANES = 16
VPR = EMBED // LANES

_mesh = plsc.VectorSubcoreMesh(core_axis_name="c", subcore_axis_name="s")


@functools.partial(
    pl.kernel,
    out_type=jax.ShapeDtypeStruct((BATCH * SEQ, EMBED), jnp.float32),
    mesh=_mesh,
    compiler_params=pltpu.CompilerParams(use_tc_tiling_on_sc=False),
    scratch_types=[
        pltpu.VMEM((ROWS_PER_W,), jnp.int32),
        pltpu.VMEM((2 * SEQ, EMBED), jnp.float32),
        pltpu.VMEM((2 * SEQ, EMBED), jnp.float32),
        pltpu.SemaphoreType.DMA,
    ],
)
def _seq_embed(seq_hbm, tok_hbm, pos_hbm, out_hbm, idx_v, pos_v, buf, sem):
    wid = lax.axis_index("s") * NC + lax.axis_index("c")
    base = wid * ROWS_PER_W

    pltpu.sync_copy(seq_hbm.at[pl.ds(base, ROWS_PER_W)], idx_v)
    pltpu.sync_copy(pos_hbm, pos_v.at[pl.ds(0, SEQ)])
    pltpu.sync_copy(pos_hbm, pos_v.at[pl.ds(SEQ, SEQ)])

    def chunk(c, carry):
        row0 = c * 2 * SEQ
        waits = []
        for o, n in ((0, 128), (128, 128), (256, 128), (384, 16)):
            waits.append(pltpu.async_copy(
                tok_hbm.at[idx_v.at[pl.ds(row0 + o, n)]],
                buf.at[pl.ds(o, n)], sem))
        for w in waits:
            w.wait()

        def add_row(j, carry2):
            for k in range(VPR):
                plsc.addupdate(
                    buf.at[j, pl.ds(k * LANES, LANES)],
                    pos_v[j, pl.ds(k * LANES, LANES)])
            return carry2

        lax.fori_loop(0, 2 * SEQ, add_row, 0, unroll=2)

        pltpu.sync_copy(buf, out_hbm.at[pl.ds(base + row0, 2 * SEQ)])
        return carry

    lax.fori_loop(0, SEQS_PER_W // 2, chunk, 0)


def kernel(sequence, token_table, pos_table):
    seq_flat = sequence.reshape(-1).astype(jnp.int32)
    out = _seq_embed(seq_flat, token_table, pos_table)
    return out.reshape(BATCH, SEQ, EMBED)


# speedup vs baseline: 1.7363x; 1.0197x over previous
import functools

import jax
import jax.numpy as jnp
from jax import lax
from jax.experimental import pallas as pl
from jax.experimental.pallas import tpu as pltpu
from jax.experimental.pallas import tpu_sc as plsc

VOCAB = 1000000
SEQ = 200
EMBED = 64
BATCH = 4096

NC = 2
NS = 16
NW = NC * NS
SEQS_PER_W = BATCH // NW
ROWS_PER_W = SEQS_PER_W * SEQ
LANES = 16
VPR = EMBED // LANES

_mesh = plsc.VectorSubcoreMesh(core_axis_name="c", subcore_axis_name="s")


@functools.partial(
    pl.kernel,
    out_type=jax.ShapeDtypeStruct((BATCH * SEQ, EMBED), jnp.float32),
    mesh=_mesh,
    compiler_params=pltpu.CompilerParams(use_tc_tiling_on_sc=False),
    scratch_types=[
        pltpu.VMEM((ROWS_PER_W,), jnp.int32),
        pltpu.VMEM((2 * SEQ, EMBED), jnp.float32),
        pltpu.VMEM((4 * SEQ, EMBED), jnp.float32),
        pltpu.SemaphoreType.DMA,
    ],
)
def _seq_embed(seq_hbm, tok_hbm, pos_hbm, out_hbm, idx_v, pos_v, buf, sem):
    wid = lax.axis_index("s") * NC + lax.axis_index("c")
    base = wid * ROWS_PER_W

    pltpu.sync_copy(seq_hbm.at[pl.ds(base, ROWS_PER_W)], idx_v)
    pltpu.sync_copy(pos_hbm, pos_v.at[pl.ds(0, SEQ)])
    pltpu.sync_copy(pos_hbm, pos_v.at[pl.ds(SEQ, SEQ)])

    def chunk(c, carry):
        row0 = c * 4 * SEQ
        waits = []
        for o in range(0, 4 * SEQ, 128):
            n = min(128, 4 * SEQ - o)
            waits.append(pltpu.async_copy(
                tok_hbm.at[idx_v.at[pl.ds(row0 + o, n)]],
                buf.at[pl.ds(o, n)], sem))
        for w in waits:
            w.wait()

        def add_row(j, carry2):
            for h in range(2):
                for k in range(VPR):
                    plsc.addupdate(
                        buf.at[h * 2 * SEQ + j, pl.ds(k * LANES, LANES)],
                        pos_v[j, pl.ds(k * LANES, LANES)])
            return carry2

        lax.fori_loop(0, 2 * SEQ, add_row, 0, unroll=2)

        pltpu.sync_copy(buf, out_hbm.at[pl.ds(base + row0, 4 * SEQ)])
        return carry

    lax.fori_loop(0, SEQS_PER_W // 4, chunk, 0)


def kernel(sequence, token_table, pos_table):
    seq_flat = sequence.reshape(-1).astype(jnp.int32)
    out = _seq_embed(seq_flat, token_table, pos_table)
    return out.reshape(BATCH, SEQ, EMBED)
